# baseline (speedup 1.0000x reference)
import jax
import jax.numpy as jnp
from jax.experimental import pallas as pl
from jax.experimental.pallas import tpu as pltpu

_ROWS = 1000000
_DIM = 64
_NCHUNK = 8
_CHUNK = _ROWS // _NCHUNK


def _body(idx_ref, emb_ref, new_ref, out_ref, sem, row_sem):
    cp = pltpu.make_async_copy(emb_ref, out_ref, sem)
    cp.start()
    cp.wait()
    idx = idx_ref[0]
    rcp = pltpu.make_async_copy(new_ref, out_ref.at[pl.ds(idx, 1), :], row_sem)
    rcp.start()
    rcp.wait()


def kernel(embeddings, emb_index, new_emb):
    idx = jnp.asarray(emb_index, jnp.int32).reshape(1)
    new_row = new_emb.reshape(1, _DIM)
    return pl.pallas_call(
        _body,
        in_specs=[
            pl.BlockSpec(memory_space=pltpu.SMEM),
            pl.BlockSpec(memory_space=pl.ANY),
            pl.BlockSpec(memory_space=pl.ANY),
        ],
        out_specs=pl.BlockSpec(memory_space=pl.ANY),
        out_shape=jax.ShapeDtypeStruct((_ROWS, _DIM), embeddings.dtype),
        scratch_shapes=[pltpu.SemaphoreType.DMA, pltpu.SemaphoreType.DMA],
    )(idx, embeddings, new_row)


# speedup vs baseline: 11.8899x; 11.8899x over previous
import jax
import jax.numpy as jnp
from jax.experimental import pallas as pl
from jax.experimental.pallas import tpu as pltpu

_ROWS = 500000
_DIM = 128
_BLK = 20000
_GRID = _ROWS // _BLK


def _body(idx_ref, emb_ref, new_ref, out_ref):
    i = pl.program_id(0)
    out_ref[...] = emb_ref[...]
    idx = idx_ref[0]
    r = idx // 2

    @pl.when(r // _BLK == i)
    def _():
        local = r - i * _BLK
        half = idx - 2 * r
        lane = jax.lax.broadcasted_iota(jnp.int32, (1, _DIM), 1)
        lo = 64 * half
        cond = (lane >= lo) & (lane < lo + 64)
        row = emb_ref[pl.ds(local, 1), :]
        out_ref[pl.ds(local, 1), :] = jnp.where(cond, new_ref[...], row)


def kernel(embeddings, emb_index, new_emb):
    idx = jnp.asarray(emb_index, jnp.int32).reshape(1)
    new_row = new_emb.reshape(1, 64)
    new128 = jnp.concatenate([new_row, new_row], axis=1)
    emb2 = embeddings.reshape(_ROWS, _DIM)
    grid_spec = pltpu.PrefetchScalarGridSpec(
        num_scalar_prefetch=1,
        grid=(_GRID,),
        in_specs=[
            pl.BlockSpec((_BLK, _DIM), lambda i, idx_ref: (i, 0)),
            pl.BlockSpec((1, _DIM), lambda i, idx_ref: (0, 0)),
        ],
        out_specs=pl.BlockSpec((_BLK, _DIM), lambda i, idx_ref: (i, 0)),
    )
    out2 = pl.pallas_call(
        _body,
        grid_spec=grid_spec,
        out_shape=jax.ShapeDtypeStruct((_ROWS, _DIM), embeddings.dtype),
    )(idx, emb2, new128)
    return out2.reshape(1000000, 64)


# speedup vs baseline: 16.1260x; 1.3563x over previous
import jax
import jax.numpy as jnp
from jax.experimental import pallas as pl
from jax.experimental.pallas import tpu as pltpu

_ROWS = 1000000
_DIM = 64
_BLK = 20000
_GRID = _ROWS // _BLK


def _body(idx_ref, emb_ref, new_ref, out_ref):
    i = pl.program_id(0)
    out_ref[...] = emb_ref[...]
    idx = idx_ref[0]

    @pl.when(idx // _BLK == i)
    def _():
        out_ref[pl.ds(idx - i * _BLK, 1), :] = new_ref[...]


def kernel(embeddings, emb_index, new_emb):
    idx = jnp.asarray(emb_index, jnp.int32).reshape(1)
    new_row = new_emb.reshape(1, _DIM)
    grid_spec = pltpu.PrefetchScalarGridSpec(
        num_scalar_prefetch=1,
        grid=(_GRID,),
        in_specs=[
            pl.BlockSpec((_BLK, _DIM), lambda i, idx_ref: (i, 0)),
            pl.BlockSpec((1, _DIM), lambda i, idx_ref: (0, 0)),
        ],
        out_specs=pl.BlockSpec((_BLK, _DIM), lambda i, idx_ref: (i, 0)),
    )
    return pl.pallas_call(
        _body,
        grid_spec=grid_spec,
        out_shape=jax.ShapeDtypeStruct((_ROWS, _DIM), embeddings.dtype),
    )(idx, embeddings, new_row)
